# Initial kernel scaffold; baseline (speedup 1.0000x reference)
import functools

import jax
import jax.numpy as jnp
from jax import lax
from jax.experimental import pallas as pl
from jax.experimental.pallas import tpu as pltpu
from jax.experimental.pallas import tpu_sc as plsc

NC = 2
NS = 16
NW = NC * NS
L = 16

D = 128
CHUNK = 128


def _make_sc_call(n_edges):
    n_chunks = n_edges // CHUNK
    assert n_chunks * CHUNK == n_edges

    mesh = plsc.VectorSubcoreMesh(
        core_axis_name="c", subcore_axis_name="s",
        num_cores=NC, num_subcores=NS)

    @functools.partial(
        pl.kernel,
        out_type=jax.ShapeDtypeStruct((n_edges,), jnp.float32),
        mesh=mesh,
        scratch_types=[
            pltpu.VMEM((CHUNK,), jnp.int32),
            pltpu.VMEM((CHUNK,), jnp.int32),
            pltpu.VMEM((CHUNK, D), jnp.float32),
            pltpu.VMEM((CHUNK, D), jnp.float32),
            pltpu.VMEM((CHUNK,), jnp.float32),
            pltpu.SemaphoreType.DMA,
        ],
    )
    def sc_call(xu_hbm, xb_hbm, src_hbm, dst_hbm, out_hbm,
                idx_u, idx_b, rows_u, rows_b, out_v, sem):
        cid = lax.axis_index("c")
        sid = lax.axis_index("s")
        wid = sid * NC + cid
        my_chunks = n_chunks // NW + jnp.where(wid < n_chunks % NW, 1, 0)

        def chunk_body(i, carry):
            off = (i * NW + wid) * CHUNK
            pltpu.sync_copy(src_hbm.at[pl.ds(off, CHUNK)], idx_u)
            pltpu.sync_copy(dst_hbm.at[pl.ds(off, CHUNK)], idx_b)
            cp_u = pltpu.async_copy(xu_hbm.at[idx_u], rows_u, sem)
            cp_b = pltpu.async_copy(xb_hbm.at[idx_b], rows_b, sem)
            cp_u.wait()
            cp_b.wait()

            def edge_body(e, c2):
                acc = rows_u[e, pl.ds(0, L)] * rows_b[e, pl.ds(0, L)]
                for d in range(1, D // L):
                    acc = acc + (rows_u[e, pl.ds(d * L, L)]
                                 * rows_b[e, pl.ds(d * L, L)])
                out_v[e] = jnp.sum(acc)
                return c2

            lax.fori_loop(0, CHUNK, edge_body, 0, unroll=4)
            pltpu.sync_copy(out_v, out_hbm.at[pl.ds(off, CHUNK)])
            return carry

        lax.fori_loop(0, my_chunks, chunk_body, 0)

    return sc_call


def kernel(x_user, x_business, edge_label_index):
    n_edges = edge_label_index.shape[1]
    idx = edge_label_index.astype(jnp.int32)
    sc_call = _make_sc_call(n_edges)
    return sc_call(x_user, x_business, idx[0], idx[1])


# speedup vs baseline: 3.0569x; 3.0569x over previous
import functools

import jax
import jax.numpy as jnp
from jax import lax
from jax.experimental import pallas as pl
from jax.experimental.pallas import tpu as pltpu
from jax.experimental.pallas import tpu_sc as plsc

NC = 2
NS = 16
NW = NC * NS
L = 16

D = 128
CHUNK = 128


def _make_sc_call(n_edges):
    n_chunks = n_edges // CHUNK
    assert n_chunks * CHUNK == n_edges

    mesh = plsc.VectorSubcoreMesh(
        core_axis_name="c", subcore_axis_name="s",
        num_cores=NC, num_subcores=NS)

    @functools.partial(
        pl.kernel,
        out_type=jax.ShapeDtypeStruct((n_edges,), jnp.float32),
        mesh=mesh,
        scratch_types=[
            pltpu.VMEM((CHUNK,), jnp.int32),
            pltpu.VMEM((CHUNK,), jnp.int32),
            pltpu.VMEM((CHUNK, D), jnp.float32),
            pltpu.VMEM((CHUNK, D), jnp.float32),
            pltpu.VMEM((CHUNK,), jnp.float32),
            pltpu.SemaphoreType.DMA,
        ],
    )
    def sc_call(xu_hbm, xb_hbm, src_hbm, dst_hbm, out_hbm,
                idx_u, idx_b, rows_u, rows_b, out_v, sem):
        cid = lax.axis_index("c")
        sid = lax.axis_index("s")
        wid = sid * NC + cid
        my_chunks = n_chunks // NW + jnp.where(wid < n_chunks % NW, 1, 0)

        lane = lax.iota(jnp.int32, 16)
        perms = [(lane + sh) % 16 for sh in (8, 4, 2, 1)]

        def chunk_body(i, carry):
            off = (i * NW + wid) * CHUNK
            pltpu.sync_copy(src_hbm.at[pl.ds(off, CHUNK)], idx_u)
            pltpu.sync_copy(dst_hbm.at[pl.ds(off, CHUNK)], idx_b)
            cp_u = pltpu.async_copy(xu_hbm.at[idx_u], rows_u, sem)
            cp_b = pltpu.async_copy(xb_hbm.at[idx_b], rows_b, sem)
            cp_u.wait()
            cp_b.wait()

            def group_body(g, c2):
                res = jnp.zeros((L,), jnp.float32)
                for j in range(L):
                    e = g * L + j
                    acc = rows_u[e, pl.ds(0, L)] * rows_b[e, pl.ds(0, L)]
                    for d in range(1, D // L):
                        acc = acc + (rows_u[e, pl.ds(d * L, L)]
                                     * rows_b[e, pl.ds(d * L, L)])
                    for p in perms:
                        acc = acc + acc.at[p].get(mode="promise_in_bounds")
                    res = jnp.where(lane == j, acc, res)
                out_v[pl.ds(g * L, L)] = res
                return c2

            lax.fori_loop(0, CHUNK // L, group_body, 0)
            pltpu.sync_copy(out_v, out_hbm.at[pl.ds(off, CHUNK)])
            return carry

        lax.fori_loop(0, my_chunks, chunk_body, 0)

    return sc_call


def kernel(x_user, x_business, edge_label_index):
    n_edges = edge_label_index.shape[1]
    idx = edge_label_index.astype(jnp.int32)
    sc_call = _make_sc_call(n_edges)
    return sc_call(x_user, x_business, idx[0], idx[1])


# speedup vs baseline: 5.0211x; 1.6425x over previous
import functools

import jax
import jax.numpy as jnp
from jax import lax
from jax.experimental import pallas as pl
from jax.experimental.pallas import tpu as pltpu
from jax.experimental.pallas import tpu_sc as plsc

NC = 2
NS = 16
NW = NC * NS
L = 16

D = 128
CHUNK = 80


def _make_sc_call(n_edges):
    e_per = n_edges // NW
    n_ch = e_per // CHUNK
    assert e_per * NW == n_edges and n_ch * CHUNK == e_per and n_ch % 2 == 1

    mesh = plsc.VectorSubcoreMesh(
        core_axis_name="c", subcore_axis_name="s",
        num_cores=NC, num_subcores=NS)

    @functools.partial(
        pl.kernel,
        out_type=jax.ShapeDtypeStruct((n_edges,), jnp.float32),
        mesh=mesh,
        scratch_types=[
            pltpu.VMEM((e_per,), jnp.int32),
            pltpu.VMEM((e_per,), jnp.int32),
            pltpu.VMEM((CHUNK, D), jnp.float32),
            pltpu.VMEM((CHUNK, D), jnp.float32),
            pltpu.VMEM((CHUNK, D), jnp.float32),
            pltpu.VMEM((CHUNK, D), jnp.float32),
            pltpu.VMEM((e_per,), jnp.float32),
            pltpu.SemaphoreType.DMA,
            pltpu.SemaphoreType.DMA,
            pltpu.SemaphoreType.DMA,
            pltpu.SemaphoreType.DMA,
        ],
    )
    def sc_call(xu_hbm, xb_hbm, src_hbm, dst_hbm, out_hbm,
                idx_u, idx_b, ru_a, rb_a, ru_b, rb_b, out_v,
                sem_ua, sem_ba, sem_ub, sem_bb):
        cid = lax.axis_index("c")
        sid = lax.axis_index("s")
        wid = sid * NC + cid
        base = wid * e_per

        lane = lax.iota(jnp.int32, 16)
        perms = [(lane + sh) % 16 for sh in (8, 4, 2, 1)]

        pltpu.sync_copy(src_hbm.at[pl.ds(base, e_per)], idx_u)
        pltpu.sync_copy(dst_hbm.at[pl.ds(base, e_per)], idx_b)

        def gathers(c, ru, rb, su, sb):
            iu = idx_u.at[pl.ds(c * CHUNK, CHUNK)]
            ib = idx_b.at[pl.ds(c * CHUNK, CHUNK)]
            return (pltpu.make_async_copy(xu_hbm.at[iu], ru, su),
                    pltpu.make_async_copy(xb_hbm.at[ib], rb, sb))

        def issue(c, ru, rb, su, sb):
            for cp in gathers(c, ru, rb, su, sb):
                cp.start()

        def wait(c, ru, rb, su, sb):
            for cp in gathers(c, ru, rb, su, sb):
                cp.wait()

        def compute(c, ru, rb):
            def group_body(g, carry):
                res = jnp.zeros((L,), jnp.float32)
                for j in range(L):
                    e = g * L + j
                    acc = ru[e, pl.ds(0, L)] * rb[e, pl.ds(0, L)]
                    for d in range(1, D // L):
                        acc = acc + (ru[e, pl.ds(d * L, L)]
                                     * rb[e, pl.ds(d * L, L)])
                    for p in perms:
                        acc = acc + acc.at[p].get(mode="promise_in_bounds")
                    res = jnp.where(lane == j, acc, res)
                out_v[pl.ds(c * CHUNK + g * L, L)] = res
                return carry

            lax.fori_loop(0, CHUNK // L, group_body, 0)

        buf_a = (ru_a, rb_a, sem_ua, sem_ba)
        buf_b = (ru_b, rb_b, sem_ub, sem_bb)

        issue(0, *buf_a)

        def pair_body(p, carry):
            c0 = 2 * p
            issue(c0 + 1, *buf_b)
            wait(c0, *buf_a)
            compute(c0, ru_a, rb_a)
            issue(c0 + 2, *buf_a)
            wait(c0 + 1, *buf_b)
            compute(c0 + 1, ru_b, rb_b)
            return carry

        lax.fori_loop(0, (n_ch - 1) // 2, pair_body, 0)

        last = n_ch - 1
        wait(last, *buf_a)
        compute(last, ru_a, rb_a)

        pltpu.sync_copy(out_v, out_hbm.at[pl.ds(base, e_per)])

    return sc_call


def kernel(x_user, x_business, edge_label_index):
    n_edges = edge_label_index.shape[1]
    idx = edge_label_index.astype(jnp.int32)
    sc_call = _make_sc_call(n_edges)
    return sc_call(x_user, x_business, idx[0], idx[1])
